# Initial kernel scaffold; baseline (speedup 1.0000x reference)
import jax
import jax.numpy as jnp
from jax.experimental import pallas as pl


def kernel(h, edge_index):
    raise NotImplementedError("write your pallas kernel here")


# speedup vs baseline: 4.2654x; 4.2654x over previous
import dataclasses
import functools

import jax
import jax.numpy as jnp
from jax import lax
from jax.experimental import pallas as pl
from jax.experimental.pallas import tpu as pltpu
from jax.experimental.pallas import tpu_sc as plsc

N_WORKERS = 32
LANES = 16
D_FEAT = 128
CHUNK = 400


@functools.cache
def _edge_dot_fn(E: int):
    epw = E // N_WORKERS
    n_chunks = epw // CHUNK
    assert epw % CHUNK == 0 and CHUNK % LANES == 0 and epw % 8 == 0

    mesh = plsc.VectorSubcoreMesh(core_axis_name="c", subcore_axis_name="s")

    cp = pltpu.CompilerParams()
    if "needs_layout_passes" in pltpu.CompilerParams.__dataclass_fields__:
        cp = dataclasses.replace(cp, needs_layout_passes=False)

    @functools.partial(
        pl.kernel,
        compiler_params=cp,
        out_type=jax.ShapeDtypeStruct((E,), jnp.float32),
        mesh=mesh,
        scratch_types=[
            pltpu.VMEM((CHUNK,), jnp.int32),
            pltpu.VMEM((CHUNK,), jnp.int32),
            pltpu.VMEM((CHUNK, D_FEAT), jnp.float32),
            pltpu.VMEM((CHUNK, D_FEAT), jnp.float32),
            pltpu.VMEM((CHUNK,), jnp.float32),
            pltpu.VMEM((LANES, LANES), jnp.float32),
            pltpu.SemaphoreType.DMA,
            pltpu.SemaphoreType.DMA,
        ],
    )
    def kern(h_hbm, src_hbm, dst_hbm, out_hbm,
             sidx, didx, u_v, v_v, out_v, acc_v, sem_u, sem_v):
        wid = lax.axis_index("s") * 2 + lax.axis_index("c")
        base = wid * epw

        @pl.loop(0, n_chunks)
        def _(j):
            off = base + j * CHUNK
            pltpu.sync_copy(src_hbm.at[pl.ds(off, CHUNK)], sidx)
            pltpu.sync_copy(dst_hbm.at[pl.ds(off, CHUNK)], didx)
            cu = pltpu.async_copy(h_hbm.at[sidx], u_v, sem_u)
            cv = pltpu.async_copy(h_hbm.at[didx], v_v, sem_v)
            cu.wait()
            cv.wait()

            @pl.loop(0, CHUNK, step=LANES)
            def _(g):
                for e in range(LANES):
                    a = (u_v[g + e, pl.ds(0, LANES)]
                         * v_v[g + e, pl.ds(0, LANES)])
                    for s_ in range(1, D_FEAT // LANES):
                        a += (u_v[g + e, pl.ds(s_ * LANES, LANES)]
                              * v_v[g + e, pl.ds(s_ * LANES, LANES)])
                    acc_v[e] = a
                rows_i = lax.iota(jnp.int32, LANES)
                s_vec = jnp.zeros((LANES,), jnp.float32)
                for f in range(LANES):
                    cols_i = jnp.full((LANES,), f, jnp.int32)
                    s_vec += plsc.load_gather(acc_v, [rows_i, cols_i])
                out_v[pl.ds(g, LANES)] = s_vec

            pltpu.sync_copy(out_v, out_hbm.at[pl.ds(off, CHUNK)])

    return kern


def kernel(h, edge_index):
    src = edge_index[0].astype(jnp.int32)
    dst = edge_index[1].astype(jnp.int32)
    return _edge_dot_fn(edge_index.shape[1])(h, src, dst)
